# baseline (speedup 1.0000x reference)
import functools

import jax
import jax.numpy as jnp
from jax import lax
from jax.experimental import pallas as pl
from jax.experimental.pallas import tpu as pltpu
from jax.experimental.pallas import tpu_sc as plsc

N = 10000
E = 320000
K = 112
NC, NS = 2, 16
NW = NC * NS
CPW = 90
EP = NW * CPW * K
RF = 632
RL = N - (NS - 1) * RF


def _sc_mesh():
    return plsc.VectorSubcoreMesh(core_axis_name="c", subcore_axis_name="s")


def _deg_call(dst3, zeros):
    @functools.partial(
        pl.kernel,
        out_type=jax.ShapeDtypeStruct((NW * N,), jnp.float32),
        mesh=_sc_mesh(),
        compiler_params=pltpu.CompilerParams(needs_layout_passes=False,
                                             use_tc_tiling_on_sc=False),
        scratch_types=[
            pltpu.VMEM((CPW, K), jnp.int32),
            pltpu.VMEM((N + 16,), jnp.float32),
        ],
    )
    def deg(dst_hbm, zeros_hbm, out_hbm, idx_v, deg_v):
        c = lax.axis_index("c")
        s = lax.axis_index("s")
        wid = s * NC + c
        def zero(i, carry):
            deg_v[pl.ds(i * 16, 16)] = jnp.zeros((16,), jnp.float32)
            return carry
        lax.fori_loop(0, (N + 16) // 16, zero, 0)
        pltpu.sync_copy(dst_hbm.at[wid], idx_v)
        ones = jnp.full((16,), 1.0, jnp.float32)

        def body(j, carry):
            def inner(q, carry2):
                v = idx_v[j, pl.ds(q * 16, 16)]
                plsc.addupdate_scatter(deg_v, [v], ones)
                return carry2
            return lax.fori_loop(0, K // 16, inner, carry)

        lax.fori_loop(0, CPW, body, 0)
        pltpu.sync_copy(deg_v.at[pl.ds(0, N)], out_hbm.at[pl.ds(wid * N, N)])

    return deg(dst3, zeros)


def _agg_call(u, src3, dst3, zeros, D=128, tc_tiling=True):
    @functools.partial(
        pl.kernel,
        out_type=jax.ShapeDtypeStruct((NC, N, D), jnp.float32),
        mesh=_sc_mesh(),
        compiler_params=pltpu.CompilerParams(use_tc_tiling_on_sc=False),
        scratch_types=[
            pltpu.VMEM((CPW * K,), jnp.int32),
            pltpu.VMEM((CPW, K), jnp.int32),
            pltpu.VMEM((K, D), jnp.float32),
            pltpu.VMEM((K, D), jnp.float32),
            pltpu.VMEM_SHARED((N + 8, D), jnp.float32),
            pltpu.SemaphoreType.DMA,
            pltpu.SemaphoreType.DMA,
        ],
    )
    def agg(u_hbm, src_hbm, dst_hbm, zeros_hbm, out_hbm,
            src_v, dst_v, rows_a, rows_b, acc_sp, sem_a, sem_b):
        c = lax.axis_index("c")
        s = lax.axis_index("s")
        wid = s * NC + c

        @pl.when(s < NS - 1)
        def _():
            pltpu.sync_copy(zeros_hbm.at[pl.ds(0, RF)],
                            acc_sp.at[pl.ds(s * RF, RF)])

        @pl.when(s == NS - 1)
        def _():
            pltpu.sync_copy(zeros_hbm.at[pl.ds(0, RL)],
                            acc_sp.at[pl.ds((NS - 1) * RF, RL)])

        pltpu.sync_copy(src_hbm.at[pl.ds(wid * CPW * K, CPW * K)], src_v)
        pltpu.sync_copy(dst_hbm.at[wid], dst_v)
        plsc.subcore_barrier()

        def gather(j, buf, sem):
            return pltpu.async_copy(
                u_hbm.at[src_v.at[pl.ds(j * K, K)]], buf, sem)

        def wait_gather(buf, sem):
            pltpu.make_async_copy(
                u_hbm.at[src_v.at[pl.ds(0, K)]], buf, sem).wait()

        gather(0, rows_a, sem_a)

        def body(i, carry):
            j = 2 * i
            gather(j + 1, rows_b, sem_b)
            wait_gather(rows_a, sem_a)
            pltpu.sync_copy(rows_a, acc_sp.at[dst_v.at[j]], add=True)

            @pl.when(j + 2 < CPW)
            def _():
                gather(j + 2, rows_a, sem_a)

            wait_gather(rows_b, sem_b)
            pltpu.sync_copy(rows_b, acc_sp.at[dst_v.at[j + 1]], add=True)
            return carry

        lax.fori_loop(0, (CPW + 1) // 2, body, 0)
        if CPW % 2 == 1:
            wait_gather(rows_a, sem_a)
            pltpu.sync_copy(rows_a, acc_sp.at[dst_v.at[CPW - 1]], add=True)
        plsc.subcore_barrier()

        @pl.when(s < NS - 1)
        def _():
            pltpu.sync_copy(acc_sp.at[pl.ds(s * RF, RF)],
                            out_hbm.at[c, pl.ds(s * RF, RF)])

        @pl.when(s == NS - 1)
        def _():
            pltpu.sync_copy(acc_sp.at[pl.ds((NS - 1) * RF, RL)],
                            out_hbm.at[c, pl.ds((NS - 1) * RF, RL)])

    return agg(u, src3, dst3, zeros)


def _tc_prep(degp, x):
    BN = 2000

    def body(degp_ref, x_ref, dis_ref, u1_ref):
        deg = jnp.sum(degp_ref[...], axis=1, keepdims=True) + 1.0
        dis = lax.rsqrt(deg)
        dis_ref[...] = dis
        u1_ref[...] = x_ref[...] * dis

    return pl.pallas_call(
        body,
        grid=(N // BN,),
        in_specs=[
            pl.BlockSpec((BN, NW), lambda i: (i, 0)),
            pl.BlockSpec((BN, 128), lambda i: (i, 0)),
        ],
        out_specs=[
            pl.BlockSpec((BN, 1), lambda i: (i, 0)),
            pl.BlockSpec((BN, 128), lambda i: (i, 0)),
        ],
        out_shape=[
            jax.ShapeDtypeStruct((N, 1), jnp.float32),
            jax.ShapeDtypeStruct((N, 128), jnp.float32),
        ],
    )(degp, x)


def _tc_mid(p, u1, dis, W1, b1r, W2p):
    BN = 1000

    def body(p_ref, u1_ref, dis_ref, W1_ref, b1_ref, W2_ref, u2_ref):
        z1 = (p_ref[0] + p_ref[1] + u1_ref[...]) * dis_ref[...]
        h1 = jnp.dot(z1, W1_ref[...], preferred_element_type=jnp.float32)
        h1 = jnp.maximum(h1 + b1_ref[...], 0.0)
        g = jnp.dot(h1, W2_ref[...], preferred_element_type=jnp.float32)
        u2_ref[...] = g * dis_ref[...]

    return pl.pallas_call(
        body,
        grid=(N // BN,),
        in_specs=[
            pl.BlockSpec((NC, BN, 128), lambda i: (0, i, 0)),
            pl.BlockSpec((BN, 128), lambda i: (i, 0)),
            pl.BlockSpec((BN, 1), lambda i: (i, 0)),
            pl.BlockSpec((128, 256), lambda i: (0, 0)),
            pl.BlockSpec((1, 256), lambda i: (0, 0)),
            pl.BlockSpec((256, 48), lambda i: (0, 0)),
        ],
        out_specs=pl.BlockSpec((BN, 48), lambda i: (i, 0)),
        out_shape=jax.ShapeDtypeStruct((N, 48), jnp.float32),
    )(p, u1, dis, W1, b1r, W2p)


def _tc_final(q, u2, dis, b2r):
    BN = 1000

    def body(q_ref, u2_ref, dis_ref, b2_ref, out_ref):
        z = (q_ref[0] + q_ref[1] + u2_ref[...]) * dis_ref[...]
        logits = z[:, :40] + b2_ref[...]
        m = jnp.max(logits, axis=1, keepdims=True)
        ex = jnp.exp(logits - m)
        lse = jnp.log(jnp.sum(ex, axis=1, keepdims=True)) + m
        out_ref[...] = logits - lse

    return pl.pallas_call(
        body,
        grid=(N // BN,),
        in_specs=[
            pl.BlockSpec((NC, BN, 48), lambda i: (0, i, 0)),
            pl.BlockSpec((BN, 48), lambda i: (i, 0)),
            pl.BlockSpec((BN, 1), lambda i: (i, 0)),
            pl.BlockSpec((1, 40), lambda i: (0, 0)),
        ],
        out_specs=pl.BlockSpec((BN, 40), lambda i: (i, 0)),
        out_shape=jax.ShapeDtypeStruct((N, 40), jnp.float32),
    )(q, u2, dis, b2r)


def kernel(x, edge, W1, b1, W2, b2):
    f32 = jnp.float32
    pad = EP - E
    src1 = jnp.concatenate([edge[0], jnp.zeros((pad,), jnp.int32)])
    dst3 = jnp.concatenate([edge[1], jnp.full((pad,), N, jnp.int32)]
                           ).reshape(NW, CPW, K)
    zeros = jnp.zeros((RF, 128), f32)
    degf = _deg_call(dst3, zeros)
    degp = degf.reshape(NW, N).T.reshape(N, NW)
    dis, u1 = _tc_prep(degp, x)
    p = _agg_call(u1, src1, dst3, zeros)
    u2 = _tc_mid(p, u1, dis, W1, b1.reshape(1, -1),
                 jnp.pad(W2, ((0, 0), (0, 8))))
    q = _agg_call(u2, src1, dst3, jnp.zeros((RF, 48), f32),
                  D=48, tc_tiling=False)
    return _tc_final(q, u2, dis, b2.reshape(1, -1))


# speedup vs baseline: 1.1357x; 1.1357x over previous
import functools

import jax
import jax.numpy as jnp
from jax import lax
from jax.experimental import pallas as pl
from jax.experimental.pallas import tpu as pltpu
from jax.experimental.pallas import tpu_sc as plsc

N = 10000
E = 320000
K = 80
NC, NS = 2, 16
NW = NC * NS
CPW = E // K // NW
RF = 632
RL = N - (NS - 1) * RF


def _sc_mesh():
    return plsc.VectorSubcoreMesh(core_axis_name="c", subcore_axis_name="s")


def _deg_call(dst3, zeros):
    @functools.partial(
        pl.kernel,
        out_type=jax.ShapeDtypeStruct((NW * N,), jnp.float32),
        mesh=_sc_mesh(),
        compiler_params=pltpu.CompilerParams(needs_layout_passes=False),
        scratch_types=[
            pltpu.VMEM((CPW, K), jnp.int32),
            pltpu.VMEM((N,), jnp.float32),
        ],
    )
    def deg(dst_hbm, zeros_hbm, out_hbm, idx_v, deg_v):
        c = lax.axis_index("c")
        s = lax.axis_index("s")
        wid = s * NC + c
        def zero(i, carry):
            deg_v[pl.ds(i * 16, 16)] = jnp.zeros((16,), jnp.float32)
            return carry
        lax.fori_loop(0, N // 16, zero, 0)
        pltpu.sync_copy(dst_hbm.at[wid], idx_v)
        ones = jnp.full((16,), 1.0, jnp.float32)

        def body(j, carry):
            def inner(q, carry2):
                v = idx_v[j, pl.ds(q * 16, 16)]
                plsc.addupdate_scatter(deg_v, [v], ones)
                return carry2
            return lax.fori_loop(0, K // 16, inner, carry)

        lax.fori_loop(0, CPW, body, 0)
        pltpu.sync_copy(deg_v, out_hbm.at[pl.ds(wid * N, N)])

    return deg(dst3, zeros)


def _agg_call(u, src3, dst3, zeros, D=128, tc_tiling=True):
    @functools.partial(
        pl.kernel,
        out_type=jax.ShapeDtypeStruct((NC, N, D), jnp.float32),
        mesh=_sc_mesh(),
        compiler_params=pltpu.CompilerParams(use_tc_tiling_on_sc=tc_tiling),
        scratch_types=[
            pltpu.VMEM((CPW * K,), jnp.int32),
            pltpu.VMEM((CPW, K), jnp.int32),
            pltpu.VMEM((K, D), jnp.float32),
            pltpu.VMEM((K, D), jnp.float32),
            pltpu.VMEM_SHARED((N, D), jnp.float32),
            pltpu.SemaphoreType.DMA,
            pltpu.SemaphoreType.DMA,
        ],
    )
    def agg(u_hbm, src_hbm, dst_hbm, zeros_hbm, out_hbm,
            src_v, dst_v, rows_a, rows_b, acc_sp, sem_a, sem_b):
        c = lax.axis_index("c")
        s = lax.axis_index("s")
        wid = s * NC + c

        @pl.when(s < NS - 1)
        def _():
            pltpu.sync_copy(zeros_hbm.at[pl.ds(0, RF)],
                            acc_sp.at[pl.ds(s * RF, RF)])

        @pl.when(s == NS - 1)
        def _():
            pltpu.sync_copy(zeros_hbm.at[pl.ds(0, RL)],
                            acc_sp.at[pl.ds((NS - 1) * RF, RL)])

        pltpu.sync_copy(src_hbm.at[pl.ds(wid * CPW * K, CPW * K)], src_v)
        pltpu.sync_copy(dst_hbm.at[wid], dst_v)
        plsc.subcore_barrier()

        def gather(j, buf, sem):
            return pltpu.async_copy(
                u_hbm.at[src_v.at[pl.ds(j * K, K)]], buf, sem)

        def wait_gather(buf, sem):
            pltpu.make_async_copy(
                u_hbm.at[src_v.at[pl.ds(0, K)]], buf, sem).wait()

        gather(0, rows_a, sem_a)

        def body(i, carry):
            j = 2 * i
            gather(j + 1, rows_b, sem_b)
            wait_gather(rows_a, sem_a)
            pltpu.sync_copy(rows_a, acc_sp.at[dst_v.at[j]], add=True)
            gather(j + 2, rows_a, sem_a)
            wait_gather(rows_b, sem_b)
            pltpu.sync_copy(rows_b, acc_sp.at[dst_v.at[j + 1]], add=True)
            return carry

        lax.fori_loop(0, CPW // 2, body, 0)
        wait_gather(rows_a, sem_a)
        pltpu.sync_copy(rows_a, acc_sp.at[dst_v.at[CPW - 1]], add=True)
        plsc.subcore_barrier()

        @pl.when(s < NS - 1)
        def _():
            pltpu.sync_copy(acc_sp.at[pl.ds(s * RF, RF)],
                            out_hbm.at[c, pl.ds(s * RF, RF)])

        @pl.when(s == NS - 1)
        def _():
            pltpu.sync_copy(acc_sp.at[pl.ds((NS - 1) * RF, RL)],
                            out_hbm.at[c, pl.ds((NS - 1) * RF, RL)])

    return agg(u, src3, dst3, zeros)


K1 = 96
CPW1 = 105
EP1 = NW * CPW1 * K1
NBUF1 = 3


def _agg1_call(u, src1p, dst1f, zeros):
    @functools.partial(
        pl.kernel,
        out_type=jax.ShapeDtypeStruct((NC, N, 128), jnp.float32),
        mesh=_sc_mesh(),
        scratch_types=[
            pltpu.VMEM((CPW1 * K1,), jnp.int32),
            pltpu.VMEM((NBUF1, K1), jnp.int32),
            [pltpu.VMEM((K1, 128), jnp.float32) for _ in range(NBUF1)],
            pltpu.VMEM_SHARED((N + 8, 128), jnp.float32),
            [pltpu.SemaphoreType.DMA for _ in range(NBUF1)],
            [pltpu.SemaphoreType.DMA for _ in range(NBUF1)],
        ],
    )
    def agg(u_hbm, src_hbm, dst_hbm, zeros_hbm, out_hbm,
            src_v, dst_vb, rows, acc_sp, sem_g, sem_d):
        c = lax.axis_index("c")
        s = lax.axis_index("s")
        wid = s * NC + c

        @pl.when(s < NS - 1)
        def _():
            pltpu.sync_copy(zeros_hbm.at[pl.ds(0, RF)],
                            acc_sp.at[pl.ds(s * RF, RF)])

        @pl.when(s == NS - 1)
        def _():
            pltpu.sync_copy(zeros_hbm.at[pl.ds(0, RL)],
                            acc_sp.at[pl.ds((NS - 1) * RF, RL)])

        pltpu.sync_copy(src_hbm.at[pl.ds(wid * CPW1 * K1, CPW1 * K1)],
                        src_v)
        plsc.subcore_barrier()
        dbase = wid * CPW1 * K1

        def fire(j, b):
            pltpu.async_copy(u_hbm.at[src_v.at[pl.ds(j * K1, K1)]],
                             rows[b], sem_g[b])
            pltpu.async_copy(dst_hbm.at[pl.ds(dbase + j * K1, K1)],
                             dst_vb.at[b], sem_d[b])

        def wait(b):
            pltpu.make_async_copy(u_hbm.at[src_v.at[pl.ds(0, K1)]],
                                  rows[b], sem_g[b]).wait()
            pltpu.make_async_copy(dst_hbm.at[pl.ds(0, K1)],
                                  dst_vb.at[b], sem_d[b]).wait()

        def scatter(b):
            pltpu.sync_copy(rows[b], acc_sp.at[dst_vb.at[b]], add=True)

        for b in range(NBUF1 - 1):
            fire(b, b)

        def body(i, carry):
            j = i * NBUF1
            for t in range(NBUF1):
                f = j + t + NBUF1 - 1
                if t == 0:
                    fire(f, NBUF1 - 1)
                else:
                    @pl.when(f < CPW1)
                    def _(f=f, t=t):
                        fire(f, t - 1)
                wait(t)
                scatter(t)
            return carry

        lax.fori_loop(0, CPW1 // NBUF1, body, 0)
        plsc.subcore_barrier()

        @pl.when(s < NS - 1)
        def _():
            pltpu.sync_copy(acc_sp.at[pl.ds(s * RF, RF)],
                            out_hbm.at[c, pl.ds(s * RF, RF)])

        @pl.when(s == NS - 1)
        def _():
            pltpu.sync_copy(acc_sp.at[pl.ds((NS - 1) * RF, RL)],
                            out_hbm.at[c, pl.ds((NS - 1) * RF, RL)])

    return agg(u, src1p, dst1f, zeros)


def _tc_prep(degp, x):
    BN = 2000

    def body(degp_ref, x_ref, dis_ref, u1_ref):
        deg = jnp.sum(degp_ref[...], axis=1, keepdims=True) + 1.0
        dis = lax.rsqrt(deg)
        dis_ref[...] = dis
        u1_ref[...] = x_ref[...] * dis

    return pl.pallas_call(
        body,
        grid=(N // BN,),
        in_specs=[
            pl.BlockSpec((BN, NW), lambda i: (i, 0)),
            pl.BlockSpec((BN, 128), lambda i: (i, 0)),
        ],
        out_specs=[
            pl.BlockSpec((BN, 1), lambda i: (i, 0)),
            pl.BlockSpec((BN, 128), lambda i: (i, 0)),
        ],
        out_shape=[
            jax.ShapeDtypeStruct((N, 1), jnp.float32),
            jax.ShapeDtypeStruct((N, 128), jnp.float32),
        ],
    )(degp, x)


def _tc_mid(p, u1, dis, W1, b1r, W2p):
    BN = 1000

    def body(p_ref, u1_ref, dis_ref, W1_ref, b1_ref, W2_ref, u2_ref):
        z1 = (p_ref[0] + p_ref[1] + u1_ref[...]) * dis_ref[...]
        h1 = jnp.dot(z1, W1_ref[...], preferred_element_type=jnp.float32)
        h1 = jnp.maximum(h1 + b1_ref[...], 0.0)
        g = jnp.dot(h1, W2_ref[...], preferred_element_type=jnp.float32)
        u2_ref[...] = g * dis_ref[...]

    return pl.pallas_call(
        body,
        grid=(N // BN,),
        in_specs=[
            pl.BlockSpec((NC, BN, 128), lambda i: (0, i, 0)),
            pl.BlockSpec((BN, 128), lambda i: (i, 0)),
            pl.BlockSpec((BN, 1), lambda i: (i, 0)),
            pl.BlockSpec((128, 256), lambda i: (0, 0)),
            pl.BlockSpec((1, 256), lambda i: (0, 0)),
            pl.BlockSpec((256, 48), lambda i: (0, 0)),
        ],
        out_specs=pl.BlockSpec((BN, 48), lambda i: (i, 0)),
        out_shape=jax.ShapeDtypeStruct((N, 48), jnp.float32),
    )(p, u1, dis, W1, b1r, W2p)


def _tc_final(q, u2, dis, b2r):
    BN = 1000

    def body(q_ref, u2_ref, dis_ref, b2_ref, out_ref):
        z = (q_ref[0] + q_ref[1] + u2_ref[...]) * dis_ref[...]
        logits = z[:, :40] + b2_ref[...]
        m = jnp.max(logits, axis=1, keepdims=True)
        ex = jnp.exp(logits - m)
        lse = jnp.log(jnp.sum(ex, axis=1, keepdims=True)) + m
        out_ref[...] = logits - lse

    return pl.pallas_call(
        body,
        grid=(N // BN,),
        in_specs=[
            pl.BlockSpec((NC, BN, 48), lambda i: (0, i, 0)),
            pl.BlockSpec((BN, 48), lambda i: (i, 0)),
            pl.BlockSpec((BN, 1), lambda i: (i, 0)),
            pl.BlockSpec((1, 40), lambda i: (0, 0)),
        ],
        out_specs=pl.BlockSpec((BN, 40), lambda i: (i, 0)),
        out_shape=jax.ShapeDtypeStruct((N, 40), jnp.float32),
    )(q, u2, dis, b2r)


def kernel(x, edge, W1, b1, W2, b2):
    f32 = jnp.float32
    src1 = edge[0]
    dst3 = edge[1].reshape(NW, CPW, K)
    zeros = jnp.zeros((RF, 128), f32)
    degf = _deg_call(dst3, zeros)
    degp = degf.reshape(NW, N).T.reshape(N, NW)
    dis, u1 = _tc_prep(degp, x)
    pad1 = EP1 - E
    src1p = jnp.concatenate([edge[0], jnp.zeros((pad1,), jnp.int32)])
    dst1f = jnp.concatenate([edge[1], jnp.full((pad1,), N, jnp.int32)])
    p = _agg1_call(u1, src1p, dst1f, zeros)
    u2 = _tc_mid(p, u1, dis, W1, b1.reshape(1, -1),
                 jnp.pad(W2, ((0, 0), (0, 8))))
    q = _agg_call(u2, src1, dst3, jnp.zeros((RF, 48), f32),
                  D=48, tc_tiling=False)
    return _tc_final(q, u2, dis, b2.reshape(1, -1))


# speedup vs baseline: 1.1661x; 1.0268x over previous
import functools

import jax
import jax.numpy as jnp
from jax import lax
from jax.experimental import pallas as pl
from jax.experimental.pallas import tpu as pltpu
from jax.experimental.pallas import tpu_sc as plsc

N = 10000
E = 320000
K = 80
NC, NS = 2, 16
NW = NC * NS
CPW = E // K // NW
RF = 632
RL = N - (NS - 1) * RF


def _sc_mesh():
    return plsc.VectorSubcoreMesh(core_axis_name="c", subcore_axis_name="s")


def _deg_call(dst3, zeros):
    @functools.partial(
        pl.kernel,
        out_type=jax.ShapeDtypeStruct((NW * N,), jnp.float32),
        mesh=_sc_mesh(),
        compiler_params=pltpu.CompilerParams(needs_layout_passes=False),
        scratch_types=[
            pltpu.VMEM((CPW, K), jnp.int32),
            pltpu.VMEM((N,), jnp.float32),
        ],
    )
    def deg(dst_hbm, zeros_hbm, out_hbm, idx_v, deg_v):
        c = lax.axis_index("c")
        s = lax.axis_index("s")
        wid = s * NC + c
        def zero(i, carry):
            deg_v[pl.ds(i * 16, 16)] = jnp.zeros((16,), jnp.float32)
            return carry
        lax.fori_loop(0, N // 16, zero, 0)
        pltpu.sync_copy(dst_hbm.at[wid], idx_v)
        ones = jnp.full((16,), 1.0, jnp.float32)

        def body(j, carry):
            def inner(q, carry2):
                v = idx_v[j, pl.ds(q * 16, 16)]
                plsc.addupdate_scatter(deg_v, [v], ones)
                return carry2
            return lax.fori_loop(0, K // 16, inner, carry)

        lax.fori_loop(0, CPW, body, 0)
        pltpu.sync_copy(deg_v, out_hbm.at[pl.ds(wid * N, N)])

    return deg(dst3, zeros)


def _agg_call(u, src3, dst3, zeros, D=128, tc_tiling=True, K_=K, CPW_=CPW):
    @functools.partial(
        pl.kernel,
        out_type=jax.ShapeDtypeStruct((NC, N, D), jnp.float32),
        mesh=_sc_mesh(),
        compiler_params=pltpu.CompilerParams(use_tc_tiling_on_sc=tc_tiling),
        scratch_types=[
            pltpu.VMEM((CPW_ * K_,), jnp.int32),
            pltpu.VMEM((CPW_, K_), jnp.int32),
            pltpu.VMEM((K_, D), jnp.float32),
            pltpu.VMEM((K_, D), jnp.float32),
            pltpu.VMEM_SHARED((N + 8, D), jnp.float32),
            pltpu.SemaphoreType.DMA,
            pltpu.SemaphoreType.DMA,
        ],
    )
    def agg(u_hbm, src_hbm, dst_hbm, zeros_hbm, out_hbm,
            src_v, dst_v, rows_a, rows_b, acc_sp, sem_a, sem_b):
        c = lax.axis_index("c")
        s = lax.axis_index("s")
        wid = s * NC + c

        @pl.when(s < NS - 1)
        def _():
            pltpu.sync_copy(zeros_hbm.at[pl.ds(0, RF)],
                            acc_sp.at[pl.ds(s * RF, RF)])

        @pl.when(s == NS - 1)
        def _():
            pltpu.sync_copy(zeros_hbm.at[pl.ds(0, RL)],
                            acc_sp.at[pl.ds((NS - 1) * RF, RL)])

        pltpu.sync_copy(src_hbm.at[pl.ds(wid * CPW_ * K_, CPW_ * K_)], src_v)
        pltpu.sync_copy(dst_hbm.at[wid], dst_v)
        plsc.subcore_barrier()

        def gather(j, buf, sem):
            return pltpu.async_copy(
                u_hbm.at[src_v.at[pl.ds(j * K_, K_)]], buf, sem)

        def wait_gather(buf, sem):
            pltpu.make_async_copy(
                u_hbm.at[src_v.at[pl.ds(0, K_)]], buf, sem).wait()

        gather(0, rows_a, sem_a)

        def body(i, carry):
            j = 2 * i
            gather(j + 1, rows_b, sem_b)
            wait_gather(rows_a, sem_a)
            pltpu.sync_copy(rows_a, acc_sp.at[dst_v.at[j]], add=True)
            gather(j + 2, rows_a, sem_a)
            wait_gather(rows_b, sem_b)
            pltpu.sync_copy(rows_b, acc_sp.at[dst_v.at[j + 1]], add=True)
            return carry

        lax.fori_loop(0, CPW_ // 2, body, 0)
        wait_gather(rows_a, sem_a)
        pltpu.sync_copy(rows_a, acc_sp.at[dst_v.at[CPW_ - 1]], add=True)
        plsc.subcore_barrier()

        @pl.when(s < NS - 1)
        def _():
            pltpu.sync_copy(acc_sp.at[pl.ds(s * RF, RF)],
                            out_hbm.at[c, pl.ds(s * RF, RF)])

        @pl.when(s == NS - 1)
        def _():
            pltpu.sync_copy(acc_sp.at[pl.ds((NS - 1) * RF, RL)],
                            out_hbm.at[c, pl.ds((NS - 1) * RF, RL)])

    return agg(u, src3, dst3, zeros)


def _tc_prep(degp, x):
    BN = 2000

    def body(degp_ref, x_ref, dis_ref, u1_ref):
        deg = jnp.sum(degp_ref[...], axis=1, keepdims=True) + 1.0
        dis = lax.rsqrt(deg)
        dis_ref[...] = dis
        u1_ref[...] = x_ref[...] * dis

    return pl.pallas_call(
        body,
        grid=(N // BN,),
        in_specs=[
            pl.BlockSpec((BN, NW), lambda i: (i, 0)),
            pl.BlockSpec((BN, 128), lambda i: (i, 0)),
        ],
        out_specs=[
            pl.BlockSpec((BN, 1), lambda i: (i, 0)),
            pl.BlockSpec((BN, 128), lambda i: (i, 0)),
        ],
        out_shape=[
            jax.ShapeDtypeStruct((N, 1), jnp.float32),
            jax.ShapeDtypeStruct((N, 128), jnp.float32),
        ],
    )(degp, x)


def _tc_mid(p, u1, dis, W1, b1r, W2p):
    BN = 1000

    def body(p_ref, u1_ref, dis_ref, W1_ref, b1_ref, W2_ref, u2_ref):
        z1 = (p_ref[0] + p_ref[1] + u1_ref[...]) * dis_ref[...]
        h1 = jnp.dot(z1, W1_ref[...], preferred_element_type=jnp.float32)
        h1 = jnp.maximum(h1 + b1_ref[...], 0.0)
        g = jnp.dot(h1, W2_ref[...], preferred_element_type=jnp.float32)
        u2_ref[...] = g * dis_ref[...]

    return pl.pallas_call(
        body,
        grid=(N // BN,),
        in_specs=[
            pl.BlockSpec((NC, BN, 128), lambda i: (0, i, 0)),
            pl.BlockSpec((BN, 128), lambda i: (i, 0)),
            pl.BlockSpec((BN, 1), lambda i: (i, 0)),
            pl.BlockSpec((128, 256), lambda i: (0, 0)),
            pl.BlockSpec((1, 256), lambda i: (0, 0)),
            pl.BlockSpec((256, 48), lambda i: (0, 0)),
        ],
        out_specs=pl.BlockSpec((BN, 48), lambda i: (i, 0)),
        out_shape=jax.ShapeDtypeStruct((N, 48), jnp.float32),
    )(p, u1, dis, W1, b1r, W2p)


def _tc_final(q, u2, dis, b2r):
    BN = 1000

    def body(q_ref, u2_ref, dis_ref, b2_ref, out_ref):
        z = (q_ref[0] + q_ref[1] + u2_ref[...]) * dis_ref[...]
        logits = z[:, :40] + b2_ref[...]
        m = jnp.max(logits, axis=1, keepdims=True)
        ex = jnp.exp(logits - m)
        lse = jnp.log(jnp.sum(ex, axis=1, keepdims=True)) + m
        out_ref[...] = logits - lse

    return pl.pallas_call(
        body,
        grid=(N // BN,),
        in_specs=[
            pl.BlockSpec((NC, BN, 48), lambda i: (0, i, 0)),
            pl.BlockSpec((BN, 48), lambda i: (i, 0)),
            pl.BlockSpec((BN, 1), lambda i: (i, 0)),
            pl.BlockSpec((1, 40), lambda i: (0, 0)),
        ],
        out_specs=pl.BlockSpec((BN, 40), lambda i: (i, 0)),
        out_shape=jax.ShapeDtypeStruct((N, 40), jnp.float32),
    )(q, u2, dis, b2r)


def kernel(x, edge, W1, b1, W2, b2):
    f32 = jnp.float32
    src1 = edge[0]
    dst3 = edge[1].reshape(NW, CPW, K)
    zeros = jnp.zeros((RF, 128), f32)
    degf = _deg_call(dst3, zeros)
    degp = degf.reshape(NW, N).T.reshape(N, NW)
    dis, u1 = _tc_prep(degp, x)
    K1, CPW1 = 104, 97
    pad1 = NW * CPW1 * K1 - E
    src1p = jnp.concatenate([edge[0], jnp.zeros((pad1,), jnp.int32)])
    dst1p = jnp.concatenate([edge[1], jnp.full((pad1,), N, jnp.int32)]
                            ).reshape(NW, CPW1, K1)
    p = _agg_call(u1, src1p, dst1p, zeros, K_=K1, CPW_=CPW1)
    u2 = _tc_mid(p, u1, dis, W1, b1.reshape(1, -1),
                 jnp.pad(W2, ((0, 0), (0, 8))))
    q = _agg_call(u2, src1, dst3, jnp.zeros((RF, 48), f32),
                  D=48, tc_tiling=False)
    return _tc_final(q, u2, dis, b2.reshape(1, -1))


# speedup vs baseline: 1.6287x; 1.3967x over previous
import functools

import jax
import jax.numpy as jnp
from jax import lax
from jax.experimental import pallas as pl
from jax.experimental.pallas import tpu as pltpu
from jax.experimental.pallas import tpu_sc as plsc

N = 10000
E = 320000
K = 80
NC, NS = 2, 16
NW = NC * NS
CPW = E // K // NW
RF = 632
RL = N - (NS - 1) * RF


def _sc_mesh():
    return plsc.VectorSubcoreMesh(core_axis_name="c", subcore_axis_name="s")


def _deg_call(dst3, zeros):
    @functools.partial(
        pl.kernel,
        out_type=jax.ShapeDtypeStruct((NW * N,), jnp.float32),
        mesh=_sc_mesh(),
        compiler_params=pltpu.CompilerParams(needs_layout_passes=False),
        scratch_types=[
            pltpu.VMEM((CPW, K), jnp.int32),
            pltpu.VMEM((N,), jnp.float32),
        ],
    )
    def deg(dst_hbm, zeros_hbm, out_hbm, idx_v, deg_v):
        c = lax.axis_index("c")
        s = lax.axis_index("s")
        wid = s * NC + c
        def zero(i, carry):
            deg_v[pl.ds(i * 16, 16)] = jnp.zeros((16,), jnp.float32)
            return carry
        lax.fori_loop(0, N // 16, zero, 0)
        pltpu.sync_copy(dst_hbm.at[wid], idx_v)
        ones = jnp.full((16,), 1.0, jnp.float32)

        def body(j, carry):
            def inner(q, carry2):
                v = idx_v[j, pl.ds(q * 16, 16)]
                plsc.addupdate_scatter(deg_v, [v], ones)
                return carry2
            return lax.fori_loop(0, K // 16, inner, carry)

        lax.fori_loop(0, CPW, body, 0)
        pltpu.sync_copy(deg_v, out_hbm.at[pl.ds(wid * N, N)])

    return deg(dst3, zeros)


def _agg_call(u, src3, dst3, zeros, D=128, tc_tiling=True, K_=K, CPW_=CPW):
    @functools.partial(
        pl.kernel,
        out_type=jax.ShapeDtypeStruct((NC, N, D), jnp.float32),
        mesh=_sc_mesh(),
        compiler_params=pltpu.CompilerParams(use_tc_tiling_on_sc=tc_tiling),
        scratch_types=[
            pltpu.VMEM((CPW_ * K_,), jnp.int32),
            pltpu.VMEM((CPW_, K_), jnp.int32),
            pltpu.VMEM((K_, D), jnp.float32),
            pltpu.VMEM((K_, D), jnp.float32),
            pltpu.VMEM_SHARED((N + 8, D), jnp.float32),
            pltpu.SemaphoreType.DMA,
            pltpu.SemaphoreType.DMA,
        ],
    )
    def agg(u_hbm, src_hbm, dst_hbm, zeros_hbm, out_hbm,
            src_v, dst_v, rows_a, rows_b, acc_sp, sem_a, sem_b):
        c = lax.axis_index("c")
        s = lax.axis_index("s")
        wid = s * NC + c

        @pl.when(s < NS - 1)
        def _():
            pltpu.sync_copy(zeros_hbm.at[pl.ds(0, RF)],
                            acc_sp.at[pl.ds(s * RF, RF)])

        @pl.when(s == NS - 1)
        def _():
            pltpu.sync_copy(zeros_hbm.at[pl.ds(0, RL)],
                            acc_sp.at[pl.ds((NS - 1) * RF, RL)])

        pltpu.sync_copy(src_hbm.at[pl.ds(wid * CPW_ * K_, CPW_ * K_)], src_v)
        pltpu.sync_copy(dst_hbm.at[wid], dst_v)
        plsc.subcore_barrier()

        def gather(j, buf, sem):
            return pltpu.async_copy(
                u_hbm.at[src_v.at[pl.ds(j * K_, K_)]], buf, sem)

        def wait_gather(buf, sem):
            pltpu.make_async_copy(
                u_hbm.at[src_v.at[pl.ds(0, K_)]], buf, sem).wait()

        gather(0, rows_a, sem_a)

        def body(i, carry):
            j = 2 * i
            gather(j + 1, rows_b, sem_b)
            wait_gather(rows_a, sem_a)
            pltpu.sync_copy(rows_a, acc_sp.at[dst_v.at[j]], add=True)
            gather(j + 2, rows_a, sem_a)
            wait_gather(rows_b, sem_b)
            pltpu.sync_copy(rows_b, acc_sp.at[dst_v.at[j + 1]], add=True)
            return carry

        lax.fori_loop(0, CPW_ // 2, body, 0)
        wait_gather(rows_a, sem_a)
        pltpu.sync_copy(rows_a, acc_sp.at[dst_v.at[CPW_ - 1]], add=True)
        plsc.subcore_barrier()

        @pl.when(s < NS - 1)
        def _():
            pltpu.sync_copy(acc_sp.at[pl.ds(s * RF, RF)],
                            out_hbm.at[c, pl.ds(s * RF, RF)])

        @pl.when(s == NS - 1)
        def _():
            pltpu.sync_copy(acc_sp.at[pl.ds((NS - 1) * RF, RL)],
                            out_hbm.at[c, pl.ds((NS - 1) * RF, RL)])

    return agg(u, src3, dst3, zeros)


NZ = N + 16
BNZ = 2504


def _tc_prep(degp, x):
    def body(degp_ref, x_ref, dis_ref, u1_ref):
        deg = jnp.sum(degp_ref[...], axis=1, keepdims=True) + 1.0
        dis = lax.rsqrt(deg)
        dis_ref[...] = dis
        u1_ref[...] = x_ref[...] * dis

        @pl.when(pl.program_id(0) == 3)
        def _():
            u1_ref[BNZ - 16:, :] = jnp.zeros((16, 128), jnp.float32)

    return pl.pallas_call(
        body,
        grid=(NZ // BNZ,),
        in_specs=[
            pl.BlockSpec((BNZ, NW), lambda i: (i, 0)),
            pl.BlockSpec((BNZ, 128), lambda i: (i, 0)),
        ],
        out_specs=[
            pl.BlockSpec((BNZ, 1), lambda i: (i, 0)),
            pl.BlockSpec((BNZ, 128), lambda i: (i, 0)),
        ],
        out_shape=[
            jax.ShapeDtypeStruct((NZ, 1), jnp.float32),
            jax.ShapeDtypeStruct((NZ, 128), jnp.float32),
        ],
    )(degp, x)


def _tc_mid(p, u1, dis, W1, b1r, W2p):
    def body(p_ref, u1_ref, dis_ref, W1_ref, b1_ref, W2_ref, u2_ref):
        z1 = (p_ref[0] + p_ref[1] + u1_ref[...]) * dis_ref[...]
        h1 = jnp.dot(z1, W1_ref[...], preferred_element_type=jnp.float32)
        h1 = jnp.maximum(h1 + b1_ref[...], 0.0)
        g = jnp.dot(h1, W2_ref[...], preferred_element_type=jnp.float32)
        u2_ref[...] = g * dis_ref[...]

        @pl.when(pl.program_id(0) == 3)
        def _():
            u2_ref[BNZ - 16:, :] = jnp.zeros((16, 48), jnp.float32)

    return pl.pallas_call(
        body,
        grid=(NZ // BNZ,),
        in_specs=[
            pl.BlockSpec((NC, BNZ, 128), lambda i: (0, i, 0)),
            pl.BlockSpec((BNZ, 128), lambda i: (i, 0)),
            pl.BlockSpec((BNZ, 1), lambda i: (i, 0)),
            pl.BlockSpec((128, 256), lambda i: (0, 0)),
            pl.BlockSpec((1, 256), lambda i: (0, 0)),
            pl.BlockSpec((256, 48), lambda i: (0, 0)),
        ],
        out_specs=pl.BlockSpec((BNZ, 48), lambda i: (i, 0)),
        out_shape=jax.ShapeDtypeStruct((NZ, 48), jnp.float32),
    )(p, u1, dis, W1, b1r, W2p)


def _tc_final(q, u2, dis, b2r):
    BN = 1000

    def body(q_ref, u2_ref, dis_ref, b2_ref, out_ref):
        z = (q_ref[0] + q_ref[1] + u2_ref[...]) * dis_ref[...]
        logits = z[:, :40] + b2_ref[...]
        m = jnp.max(logits, axis=1, keepdims=True)
        ex = jnp.exp(logits - m)
        lse = jnp.log(jnp.sum(ex, axis=1, keepdims=True)) + m
        out_ref[...] = logits - lse

    return pl.pallas_call(
        body,
        grid=(N // BN,),
        in_specs=[
            pl.BlockSpec((NC, BN, 48), lambda i: (0, i, 0)),
            pl.BlockSpec((BN, 48), lambda i: (i, 0)),
            pl.BlockSpec((BN, 1), lambda i: (i, 0)),
            pl.BlockSpec((1, 40), lambda i: (0, 0)),
        ],
        out_specs=pl.BlockSpec((BN, 40), lambda i: (i, 0)),
        out_shape=jax.ShapeDtypeStruct((N, 40), jnp.float32),
    )(q, u2, dis, b2r)


def kernel(x, edge, W1, b1, W2, b2):
    f32 = jnp.float32
    src1 = edge[0]
    dst3 = edge[1].reshape(NW, CPW, K)
    zeros = jnp.zeros((RF, 128), f32)
    degf = _deg_call(dst3, zeros)
    degp = degf.reshape(NW, N).T.reshape(N, NW)
    dis, u1 = _tc_prep(degp, x)
    K1, CPW1 = 104, 97
    pad1 = NW * CPW1 * K1 - E
    pidx = jnp.arange(pad1, dtype=jnp.int32)
    src1p = jnp.concatenate([edge[0], N + (pidx % 16)])
    dst1p = jnp.concatenate([edge[1], pidx % N]).reshape(NW, CPW1, K1)
    p = _agg_call(u1, src1p, dst1p, zeros, K_=K1, CPW_=CPW1)
    u2 = _tc_mid(p, u1, dis, W1, b1.reshape(1, -1),
                 jnp.pad(W2, ((0, 0), (0, 8))))
    q = _agg_call(u2, src1, dst3, jnp.zeros((RF, 48), f32),
                  D=48, tc_tiling=False)
    return _tc_final(q, u2, dis, b2.reshape(1, -1))


# speedup vs baseline: 1.6849x; 1.0345x over previous
import functools

import jax
import jax.numpy as jnp
from jax import lax
from jax.experimental import pallas as pl
from jax.experimental.pallas import tpu as pltpu
from jax.experimental.pallas import tpu_sc as plsc

N = 10000
E = 320000
K = 80
NC, NS = 2, 16
NW = NC * NS
CPW = E // K // NW
RF = 632
RL = N - (NS - 1) * RF


def _sc_mesh():
    return plsc.VectorSubcoreMesh(core_axis_name="c", subcore_axis_name="s")


def _deg_call(dst3, zeros):
    @functools.partial(
        pl.kernel,
        out_type=jax.ShapeDtypeStruct((NW * N,), jnp.float32),
        mesh=_sc_mesh(),
        compiler_params=pltpu.CompilerParams(needs_layout_passes=False),
        scratch_types=[
            pltpu.VMEM((CPW, K), jnp.int32),
            pltpu.VMEM((N,), jnp.float32),
        ],
    )
    def deg(dst_hbm, zeros_hbm, out_hbm, idx_v, deg_v):
        c = lax.axis_index("c")
        s = lax.axis_index("s")
        wid = s * NC + c
        def zero(i, carry):
            deg_v[pl.ds(i * 16, 16)] = jnp.zeros((16,), jnp.float32)
            return carry
        lax.fori_loop(0, N // 16, zero, 0)
        pltpu.sync_copy(dst_hbm.at[wid], idx_v)
        ones = jnp.full((16,), 1.0, jnp.float32)

        def body(j, carry):
            def inner(q, carry2):
                v = idx_v[j, pl.ds(q * 16, 16)]
                plsc.addupdate_scatter(deg_v, [v], ones)
                return carry2
            return lax.fori_loop(0, K // 16, inner, carry)

        lax.fori_loop(0, CPW, body, 0)
        pltpu.sync_copy(deg_v, out_hbm.at[pl.ds(wid * N, N)])

    return deg(dst3, zeros)


def _agg_call(u, src3, dst3, zeros, D=128, tc_tiling=True, K_=K, CPW_=CPW):
    @functools.partial(
        pl.kernel,
        out_type=jax.ShapeDtypeStruct((NC, N, D), jnp.float32),
        mesh=_sc_mesh(),
        compiler_params=pltpu.CompilerParams(use_tc_tiling_on_sc=tc_tiling),
        scratch_types=[
            pltpu.VMEM((CPW_ * K_,), jnp.int32),
            pltpu.VMEM((CPW_, K_), jnp.int32),
            pltpu.VMEM((K_, D), jnp.float32),
            pltpu.VMEM((K_, D), jnp.float32),
            pltpu.VMEM_SHARED((N + 8, D), jnp.float32),
            pltpu.SemaphoreType.DMA,
            pltpu.SemaphoreType.DMA,
        ],
    )
    def agg(u_hbm, src_hbm, dst_hbm, zeros_hbm, out_hbm,
            src_v, dst_v, rows_a, rows_b, acc_sp, sem_a, sem_b):
        c = lax.axis_index("c")
        s = lax.axis_index("s")
        wid = s * NC + c

        @pl.when(s < NS - 1)
        def _():
            pltpu.sync_copy(zeros_hbm.at[pl.ds(0, RF)],
                            acc_sp.at[pl.ds(s * RF, RF)])

        @pl.when(s == NS - 1)
        def _():
            pltpu.sync_copy(zeros_hbm.at[pl.ds(0, RL)],
                            acc_sp.at[pl.ds((NS - 1) * RF, RL)])

        pltpu.sync_copy(src_hbm.at[pl.ds(wid * CPW_ * K_, CPW_ * K_)], src_v)
        pltpu.sync_copy(dst_hbm.at[wid], dst_v)
        plsc.subcore_barrier()

        def gather(j, buf, sem):
            return pltpu.async_copy(
                u_hbm.at[src_v.at[pl.ds(j * K_, K_)]], buf, sem)

        def wait_gather(buf, sem):
            pltpu.make_async_copy(
                u_hbm.at[src_v.at[pl.ds(0, K_)]], buf, sem).wait()

        gather(0, rows_a, sem_a)

        def body(i, carry):
            j = 2 * i
            gather(j + 1, rows_b, sem_b)
            wait_gather(rows_a, sem_a)
            pltpu.sync_copy(rows_a, acc_sp.at[dst_v.at[j]], add=True)
            gather(j + 2, rows_a, sem_a)
            wait_gather(rows_b, sem_b)
            pltpu.sync_copy(rows_b, acc_sp.at[dst_v.at[j + 1]], add=True)
            return carry

        lax.fori_loop(0, CPW_ // 2, body, 0)
        wait_gather(rows_a, sem_a)
        pltpu.sync_copy(rows_a, acc_sp.at[dst_v.at[CPW_ - 1]], add=True)
        plsc.subcore_barrier()

        @pl.when(s < NS - 1)
        def _():
            pltpu.sync_copy(acc_sp.at[pl.ds(s * RF, RF)],
                            out_hbm.at[c, pl.ds(s * RF, RF)])

        @pl.when(s == NS - 1)
        def _():
            pltpu.sync_copy(acc_sp.at[pl.ds((NS - 1) * RF, RL)],
                            out_hbm.at[c, pl.ds((NS - 1) * RF, RL)])

    return agg(u, src3, dst3, zeros)


NZ = N + 16
BNZ = 2504


def _tc_prep(degp, x):
    def body(degp_ref, x_ref, dis_ref, u1_ref):
        deg = jnp.sum(degp_ref[...], axis=1, keepdims=True) + 1.0
        dis = lax.rsqrt(deg)
        dis_ref[...] = dis
        u1_ref[...] = x_ref[...] * dis

        @pl.when(pl.program_id(0) == 3)
        def _():
            u1_ref[BNZ - 16:, :] = jnp.zeros((16, 128), jnp.float32)

    return pl.pallas_call(
        body,
        grid=(NZ // BNZ,),
        in_specs=[
            pl.BlockSpec((BNZ, NW), lambda i: (i, 0)),
            pl.BlockSpec((BNZ, 128), lambda i: (i, 0)),
        ],
        out_specs=[
            pl.BlockSpec((BNZ, 1), lambda i: (i, 0)),
            pl.BlockSpec((BNZ, 128), lambda i: (i, 0)),
        ],
        out_shape=[
            jax.ShapeDtypeStruct((NZ, 1), jnp.float32),
            jax.ShapeDtypeStruct((NZ, 128), jnp.float32),
        ],
    )(degp, x)


def _tc_mid(p, u1, dis, W1, b1r, W2p):
    def body(p_ref, u1_ref, dis_ref, W1_ref, b1_ref, W2_ref, u2_ref):
        z1 = (p_ref[0] + p_ref[1] + u1_ref[...]) * dis_ref[...]
        h1 = jnp.dot(z1, W1_ref[...], preferred_element_type=jnp.float32)
        h1 = jnp.maximum(h1 + b1_ref[...], 0.0)
        g = jnp.dot(h1, W2_ref[...], preferred_element_type=jnp.float32)
        u2_ref[...] = g * dis_ref[...]

        @pl.when(pl.program_id(0) == 3)
        def _():
            u2_ref[BNZ - 16:, :] = jnp.zeros((16, 48), jnp.float32)

    return pl.pallas_call(
        body,
        grid=(NZ // BNZ,),
        in_specs=[
            pl.BlockSpec((NC, BNZ, 128), lambda i: (0, i, 0)),
            pl.BlockSpec((BNZ, 128), lambda i: (i, 0)),
            pl.BlockSpec((BNZ, 1), lambda i: (i, 0)),
            pl.BlockSpec((128, 256), lambda i: (0, 0)),
            pl.BlockSpec((1, 256), lambda i: (0, 0)),
            pl.BlockSpec((256, 48), lambda i: (0, 0)),
        ],
        out_specs=pl.BlockSpec((BNZ, 48), lambda i: (i, 0)),
        out_shape=jax.ShapeDtypeStruct((NZ, 48), jnp.float32),
    )(p, u1, dis, W1, b1r, W2p)


def _tc_final(q, u2, dis, b2r):
    BN = 1000

    def body(q_ref, u2_ref, dis_ref, b2_ref, out_ref):
        z = (q_ref[0] + q_ref[1] + u2_ref[...]) * dis_ref[...]
        logits = z[:, :40] + b2_ref[...]
        m = jnp.max(logits, axis=1, keepdims=True)
        ex = jnp.exp(logits - m)
        lse = jnp.log(jnp.sum(ex, axis=1, keepdims=True)) + m
        out_ref[...] = logits - lse

    return pl.pallas_call(
        body,
        grid=(N // BN,),
        in_specs=[
            pl.BlockSpec((NC, BN, 48), lambda i: (0, i, 0)),
            pl.BlockSpec((BN, 48), lambda i: (i, 0)),
            pl.BlockSpec((BN, 1), lambda i: (i, 0)),
            pl.BlockSpec((1, 40), lambda i: (0, 0)),
        ],
        out_specs=pl.BlockSpec((BN, 40), lambda i: (i, 0)),
        out_shape=jax.ShapeDtypeStruct((N, 40), jnp.float32),
    )(q, u2, dis, b2r)


def kernel(x, edge, W1, b1, W2, b2):
    f32 = jnp.float32
    src1 = edge[0]
    dst3 = edge[1].reshape(NW, CPW, K)
    zeros = jnp.zeros((RF, 128), f32)
    degf = _deg_call(dst3, zeros)
    degp = degf.reshape(NW, N).T.reshape(N, NW)
    dis, u1 = _tc_prep(degp, x)
    K1, CPW1 = 104, 97
    pad1 = NW * CPW1 * K1 - E
    pidx = jnp.arange(pad1, dtype=jnp.int32)
    src1p = jnp.concatenate([edge[0], N + (pidx % 16)])
    dst1p = jnp.concatenate([edge[1], pidx % N]).reshape(NW, CPW1, K1)
    p = _agg_call(u1, src1p, dst1p, zeros, K_=K1, CPW_=CPW1)
    u2 = _tc_mid(p, u1, dis, W1, b1.reshape(1, -1),
                 jnp.pad(W2, ((0, 0), (0, 8))))
    q = _agg_call(u2, src1p, dst1p, jnp.zeros((RF, 48), f32),
                  D=48, tc_tiling=False, K_=K1, CPW_=CPW1)
    return _tc_final(q, u2, dis, b2.reshape(1, -1))
